# Initial kernel scaffold; baseline (speedup 1.0000x reference)
import jax
import jax.numpy as jnp
from jax.experimental import pallas as pl


def kernel(hidden_states, gate_w, w13, w2):
    raise NotImplementedError("write your pallas kernel here")


# speedup vs baseline: 1.9691x; 1.9691x over previous
import functools

import jax
import jax.numpy as jnp
from jax import lax
from jax.experimental import pallas as pl
from jax.experimental.pallas import tpu as pltpu

NE = 64
KTOP = 8
HD = 2048
ID = 768
BT = 128
RB = 256


def _router_body(x_ref, gw_ref, logits_ref, wout_ref, iout_ref):
    x = x_ref[...]
    logits = lax.dot_general(x, gw_ref[...], (((1,), (1,)), ((), ())),
                             preferred_element_type=jnp.float32)
    logits_ref[...] = logits
    m = jnp.max(logits, axis=-1, keepdims=True)
    p = jnp.exp(logits - m)
    probs = p / jnp.sum(p, axis=-1, keepdims=True)

    iot = lax.broadcasted_iota(jnp.int32, probs.shape, 1)
    ws = []
    idxs = []
    pc = probs
    for _ in range(KTOP):
        mx = jnp.max(pc, axis=-1, keepdims=True)
        cand = jnp.where(pc == mx, iot, NE)
        idx = jnp.min(cand, axis=-1, keepdims=True)
        ws.append(mx)
        idxs.append(idx)
        pc = jnp.where(iot == idx, -1.0, pc)
    w = jnp.concatenate(ws, axis=-1)
    w = w / jnp.sum(w, axis=-1, keepdims=True)
    wout_ref[...] = w
    iout_ref[...] = jnp.concatenate(idxs, axis=-1)


def _run_router(x, gate_w):
    T = x.shape[0]
    grid = (T // RB,)
    return pl.pallas_call(
        _router_body,
        grid=grid,
        in_specs=[
            pl.BlockSpec((RB, HD), lambda i: (i, 0)),
            pl.BlockSpec((NE, HD), lambda i: (0, 0)),
        ],
        out_specs=[
            pl.BlockSpec((RB, NE), lambda i: (i, 0)),
            pl.BlockSpec((RB, KTOP), lambda i: (i, 0)),
            pl.BlockSpec((RB, KTOP), lambda i: (i, 0)),
        ],
        out_shape=[
            jax.ShapeDtypeStruct((T, NE), jnp.float32),
            jax.ShapeDtypeStruct((T, KTOP), jnp.float32),
            jax.ShapeDtypeStruct((T, KTOP), jnp.int32),
        ],
    )(x, gate_w)


def _gmm_body(be_ref, xg_ref, w13_ref, w2_ref, y_ref):
    b = pl.program_id(0)

    @pl.when(be_ref[b] >= 0)
    def _():
        xb = xg_ref[...].astype(jnp.bfloat16)
        w13 = w13_ref[0].astype(jnp.bfloat16)
        h = lax.dot_general(xb, w13, (((1,), (1,)), ((), ())),
                            preferred_element_type=jnp.float32)
        g = h[:, :ID]
        u = h[:, ID:]
        a = (g * (1.0 / (1.0 + jnp.exp(-g))) * u).astype(jnp.bfloat16)
        w2 = w2_ref[0].astype(jnp.bfloat16)
        y_ref[...] = lax.dot_general(a, w2, (((1,), (1,)), ((), ())),
                                     preferred_element_type=jnp.float32)


def _run_gmm(xg, w13, w2, block_expert, nblocks):
    P = xg.shape[0]
    grid_spec = pltpu.PrefetchScalarGridSpec(
        num_scalar_prefetch=1,
        grid=(nblocks,),
        in_specs=[
            pl.BlockSpec((BT, HD), lambda b, be: (b, 0)),
            pl.BlockSpec((1, 2 * ID, HD),
                         lambda b, be: (jnp.maximum(be[b], 0), 0, 0)),
            pl.BlockSpec((1, HD, ID),
                         lambda b, be: (jnp.maximum(be[b], 0), 0, 0)),
        ],
        out_specs=pl.BlockSpec((BT, HD), lambda b, be: (b, 0)),
    )
    return pl.pallas_call(
        _gmm_body,
        grid_spec=grid_spec,
        out_shape=jax.ShapeDtypeStruct((P, HD), jnp.float32),
    )(block_expert, xg, w13, w2)


def _build_routing(topk_i):
    TK = topk_i.size
    nblocks = TK // BT + NE
    flat_e = topk_i.reshape(-1)
    oh = (flat_e[:, None] == jnp.arange(NE, dtype=jnp.int32)[None, :])
    csum = jnp.cumsum(oh.astype(jnp.int32), axis=0)
    counts = csum[-1]
    rank = jnp.take_along_axis(csum, flat_e[:, None], axis=1)[:, 0] - 1
    nb_e = (counts + BT - 1) // BT
    cum_nb = jnp.cumsum(nb_e)
    total_blocks = cum_nb[-1]
    padded_off = BT * (cum_nb - nb_e)
    barr = jnp.arange(nblocks, dtype=jnp.int32)
    block_expert = jnp.where(
        barr < total_blocks,
        jnp.searchsorted(cum_nb, barr, side='right').astype(jnp.int32),
        -1)
    dest = padded_off[flat_e] + rank
    src_token = jnp.zeros((nblocks * BT,), jnp.int32).at[dest].set(
        jnp.arange(TK, dtype=jnp.int32) // KTOP)
    pos = dest.reshape(topk_i.shape)
    return src_token, block_expert, pos


@jax.jit
def kernel(hidden_states, gate_w, w13, w2):
    B, S, Hd = hidden_states.shape
    x = hidden_states.reshape(-1, Hd)

    logits, topk_w, topk_i = _run_router(x, gate_w)
    src_token, block_expert, pos = _build_routing(topk_i)

    nblocks = src_token.shape[0] // BT
    xg = x[src_token]
    y = _run_gmm(xg, w13, w2, block_expert, nblocks)
    out = jnp.sum(topk_w[..., None] * y[pos], axis=1)

    return out, logits


# speedup vs baseline: 2.2652x; 1.1504x over previous
import functools

import jax
import jax.numpy as jnp
from jax import lax
from jax.experimental import pallas as pl
from jax.experimental.pallas import tpu as pltpu
from jax.experimental.pallas import tpu_sc as plsc

NE = 64
KTOP = 8
HD = 2048
ID = 768
BT = 128
RB = 256


def _router_body(x_ref, gw_ref, logits_ref, wout_ref, iout_ref, cnt_ref):
    x = x_ref[...]
    logits = lax.dot_general(x, gw_ref[...], (((1,), (1,)), ((), ())),
                             preferred_element_type=jnp.float32)
    logits_ref[...] = logits
    m = jnp.max(logits, axis=-1, keepdims=True)
    p = jnp.exp(logits - m)
    probs = p / jnp.sum(p, axis=-1, keepdims=True)

    iot = lax.broadcasted_iota(jnp.int32, probs.shape, 1)
    ws = []
    idxs = []
    pc = probs
    for _ in range(KTOP):
        mx = jnp.max(pc, axis=-1, keepdims=True)
        cand = jnp.where(pc == mx, iot, NE)
        idx = jnp.min(cand, axis=-1, keepdims=True)
        ws.append(mx)
        idxs.append(idx)
        pc = jnp.where(iot == idx, -1.0, pc)
    w = jnp.concatenate(ws, axis=-1)
    w = w / jnp.sum(w, axis=-1, keepdims=True)
    wout_ref[...] = w
    idx = jnp.concatenate(idxs, axis=-1)
    iout_ref[...] = idx
    cnt = jnp.zeros((1, NE), jnp.float32)
    for k in range(KTOP):
        ohk = (idx[:, k:k + 1] ==
               lax.broadcasted_iota(jnp.int32, (RB, NE), 1)).astype(jnp.float32)
        cnt = cnt + jnp.sum(ohk, axis=0, keepdims=True)
    cnt_ref[...] = cnt.astype(jnp.int32)[None]


def _run_router(x, gate_w):
    T = x.shape[0]
    grid = (T // RB,)
    return pl.pallas_call(
        _router_body,
        grid=grid,
        in_specs=[
            pl.BlockSpec((RB, HD), lambda i: (i, 0)),
            pl.BlockSpec((NE, HD), lambda i: (0, 0)),
        ],
        out_specs=[
            pl.BlockSpec((RB, NE), lambda i: (i, 0)),
            pl.BlockSpec((RB, KTOP), lambda i: (i, 0)),
            pl.BlockSpec((RB, KTOP), lambda i: (i, 0)),
            pl.BlockSpec((1, 1, NE), lambda i: (i, 0, 0)),
        ],
        out_shape=[
            jax.ShapeDtypeStruct((T, NE), jnp.float32),
            jax.ShapeDtypeStruct((T, KTOP), jnp.float32),
            jax.ShapeDtypeStruct((T, KTOP), jnp.int32),
            jax.ShapeDtypeStruct((T // RB, 1, NE), jnp.int32),
        ],
    )(x, gate_w)


def _dest_body(i_ref, pb_ref, po_ref, dest_ref):
    idx = i_ref[...]
    base = (pb_ref[0] + po_ref[0]).astype(jnp.float32)
    tril = (lax.broadcasted_iota(jnp.int32, (RB, RB), 0) >
            lax.broadcasted_iota(jnp.int32, (RB, RB), 1)).astype(jnp.float32)
    iot = lax.broadcasted_iota(jnp.int32, (RB, NE), 1)
    for k in range(KTOP):
        ohk = (idx[:, k:k + 1] == iot).astype(jnp.float32)
        prev_t = lax.dot_general(tril, ohk, (((1,), (0,)), ((), ())),
                                 preferred_element_type=jnp.float32)
        slot = jnp.sum((prev_t + base) * ohk, axis=1)
        dest_ref[:, k] = slot.astype(jnp.int32)
        base = base + jnp.sum(ohk, axis=0, keepdims=True)


def _run_dest(topk_i, prefix_before, padded_off):
    T = topk_i.shape[0]
    return pl.pallas_call(
        _dest_body,
        grid=(T // RB,),
        in_specs=[
            pl.BlockSpec((RB, KTOP), lambda i: (i, 0)),
            pl.BlockSpec((1, 1, NE), lambda i: (i, 0, 0)),
            pl.BlockSpec((1, 1, NE), lambda i: (0, 0, 0)),
        ],
        out_specs=pl.BlockSpec((RB, KTOP), lambda i: (i, 0)),
        out_shape=jax.ShapeDtypeStruct((T, KTOP), jnp.int32),
    )(topk_i, prefix_before, padded_off)


SC_NW = 32
SC_C = 16


def _sc_dispatch_body(x3_hbm, tok_hbm, dest_hbm, xg3_hbm,
                      tok_v, dest_v, gbuf, gsems, ssems):
    wid = lax.axis_index("s") * 2 + lax.axis_index("c")
    TK = tok_hbm.shape[0]
    per_w = TK // SC_NW
    nch = per_w // SC_C
    base = wid * per_w
    pltpu.sync_copy(tok_hbm.at[pl.ds(base, per_w)], tok_v)
    pltpu.sync_copy(dest_hbm.at[pl.ds(base, per_w)], dest_v)
    pending = {}
    for j in range(nch):
        tvec = tok_v[pl.ds(j * SC_C, SC_C)]
        dvec = dest_v[pl.ds(j * SC_C, SC_C)]
        sl = j % 2
        if j >= 2:
            pending[j - 2].wait()
        g = pltpu.make_async_copy(x3_hbm.at[tvec], gbuf.at[sl], gsems[sl])
        g.start()
        g.wait()
        s = pltpu.make_async_copy(gbuf.at[sl], xg3_hbm.at[dvec], ssems[sl])
        s.start()
        pending[j] = s
    pending[nch - 2].wait()
    pending[nch - 1].wait()


def _run_sc_dispatch(x3, tok, dest, P):
    TK = tok.shape[0]
    mesh = plsc.VectorSubcoreMesh(core_axis_name="c", subcore_axis_name="s")
    f = pl.kernel(
        _sc_dispatch_body,
        out_type=jax.ShapeDtypeStruct((P, HD // 128, 128), jnp.float32),
        mesh=mesh,
        scratch_types=[
            pltpu.VMEM((TK // SC_NW,), jnp.int32),
            pltpu.VMEM((TK // SC_NW,), jnp.int32),
            pltpu.VMEM((2, SC_C, HD // 128, 128), jnp.float32),
            (pltpu.SemaphoreType.DMA, pltpu.SemaphoreType.DMA),
            (pltpu.SemaphoreType.DMA, pltpu.SemaphoreType.DMA),
        ],
    )
    return f(x3, tok, dest)


NC13 = 8
NC2 = 4
C13 = 2 * ID // NC13
C2 = HD // NC2


def _start_weight_dmas(e, slot, w13_hbm, w2_hbm, w13c, w2c, sem13, sem2):
    for c in range(NC13):
        pltpu.make_async_copy(
            w13_hbm.at[e, pl.ds(c * C13, C13)],
            w13c.at[slot, pl.ds(c * C13, C13)],
            sem13.at[slot, c]).start()
    for c in range(NC2):
        pltpu.make_async_copy(
            w2_hbm.at[e, pl.ds(c * C2, C2)],
            w2c.at[slot, pl.ds(c * C2, C2)],
            sem2.at[slot, c]).start()


def _wait_weight_dmas(e, slot, w13_hbm, w2_hbm, w13c, w2c, sem13, sem2):
    for c in range(NC13):
        pltpu.make_async_copy(
            w13_hbm.at[e, pl.ds(c * C13, C13)],
            w13c.at[slot, pl.ds(c * C13, C13)],
            sem13.at[slot, c]).wait()
    for c in range(NC2):
        pltpu.make_async_copy(
            w2_hbm.at[e, pl.ds(c * C2, C2)],
            w2c.at[slot, pl.ds(c * C2, C2)],
            sem2.at[slot, c]).wait()


def _gmm_body(be_ref, eord_ref, first_ref, nexte_ref,
              xg_ref, w13_hbm, w2_hbm, y_ref,
              w13c, w2c, w13b, w2b, sem13, sem2):
    b = pl.program_id(0)
    be = be_ref[b]

    @pl.when(be >= 0)
    def _():
        slot = eord_ref[b] & 1

        @pl.when(b == 0)
        def _():
            _start_weight_dmas(be, slot, w13_hbm, w2_hbm, w13c, w2c,
                               sem13, sem2)

        @pl.when(first_ref[b] == 1)
        def _():
            nxt = nexte_ref[b]

            @pl.when(nxt >= 0)
            def _():
                _start_weight_dmas(nxt, 1 - slot, w13_hbm, w2_hbm,
                                   w13c, w2c, sem13, sem2)

            _wait_weight_dmas(be, slot, w13_hbm, w2_hbm, w13c, w2c,
                              sem13, sem2)
            w13b[...] = w13c[slot].astype(jnp.bfloat16)
            w2b[...] = w2c[slot].astype(jnp.bfloat16)

        xb = xg_ref[...].astype(jnp.bfloat16)
        h = lax.dot_general(xb, w13b[...], (((1,), (1,)), ((), ())),
                            preferred_element_type=jnp.float32)
        g = h[:, :ID]
        u = h[:, ID:]
        a = (g * (1.0 / (1.0 + jnp.exp(-g))) * u).astype(jnp.bfloat16)
        y_ref[...] = lax.dot_general(a, w2b[...], (((1,), (1,)), ((), ())),
                                     preferred_element_type=jnp.float32
                                     ).astype(jnp.bfloat16)


def _run_gmm(xg, w13, w2, block_expert, block_eord, block_first, block_nexte,
             nblocks):
    P = xg.shape[0]
    grid_spec = pltpu.PrefetchScalarGridSpec(
        num_scalar_prefetch=4,
        grid=(nblocks,),
        in_specs=[
            pl.BlockSpec((BT, HD), lambda b, *_: (b, 0)),
            pl.BlockSpec(memory_space=pl.ANY),
            pl.BlockSpec(memory_space=pl.ANY),
        ],
        out_specs=pl.BlockSpec((BT, HD), lambda b, *_: (b, 0)),
        scratch_shapes=[
            pltpu.VMEM((2, 2 * ID, HD), jnp.float32),
            pltpu.VMEM((2, HD, ID), jnp.float32),
            pltpu.VMEM((2 * ID, HD), jnp.bfloat16),
            pltpu.VMEM((HD, ID), jnp.bfloat16),
            pltpu.SemaphoreType.DMA((2, NC13)),
            pltpu.SemaphoreType.DMA((2, NC2)),
        ],
    )
    return pl.pallas_call(
        _gmm_body,
        grid_spec=grid_spec,
        out_shape=jax.ShapeDtypeStruct((P, HD), jnp.bfloat16),
    )(block_expert, block_eord, block_first, block_nexte, xg, w13, w2)


def _build_routing(counts_blk, TK):
    nblocks = TK // BT + NE
    counts = jnp.sum(counts_blk, axis=0)
    prefix_before = jnp.cumsum(counts_blk, axis=0) - counts_blk
    nb_e = (counts + BT - 1) // BT
    cum_nb = jnp.cumsum(nb_e)
    total_blocks = cum_nb[-1]
    padded_off = BT * (cum_nb - nb_e)
    barr = jnp.arange(nblocks, dtype=jnp.int32)
    block_expert = jnp.where(
        barr < total_blocks,
        jnp.searchsorted(cum_nb, barr, side='right').astype(jnp.int32),
        -1)

    used = counts > 0
    ord_e = jnp.cumsum(used.astype(jnp.int32)) - 1
    ord_clamped = jnp.where(used, ord_e, NE)
    expert_of_ord = jnp.full((NE + 1,), -1, jnp.int32).at[ord_clamped].set(
        jnp.arange(NE, dtype=jnp.int32), mode='drop')
    next_of_e = expert_of_ord[jnp.clip(ord_e + 1, 0, NE)]
    live = block_expert >= 0
    e_safe = jnp.maximum(block_expert, 0)
    block_eord = jnp.where(live, ord_e[e_safe], 0).astype(jnp.int32)
    block_nexte = jnp.where(live, next_of_e[e_safe], -1).astype(jnp.int32)
    prev_e = jnp.concatenate([jnp.array([-2], jnp.int32), block_expert[:-1]])
    block_first = (live & (block_expert != prev_e)).astype(jnp.int32)
    return (prefix_before, padded_off, block_expert,
            block_eord, block_first, block_nexte)


@jax.jit
def kernel(hidden_states, gate_w, w13, w2):
    B, S, Hd = hidden_states.shape
    x = hidden_states.reshape(-1, Hd)
    T = x.shape[0]
    TK = T * KTOP

    logits, topk_w, topk_i, counts_blk3 = _run_router(x, gate_w)
    (prefix_before, padded_off, block_expert,
     block_eord, block_first, block_nexte) = _build_routing(
         counts_blk3[:, 0, :], TK)

    dest = _run_dest(topk_i, prefix_before[:, None, :],
                     padded_off[None, None, :])
    nblocks = TK // BT + NE
    P = nblocks * BT

    tok = (jnp.arange(TK, dtype=jnp.int32) // KTOP)
    xg3 = _run_sc_dispatch(x.reshape(T, HD // 128, 128),
                           tok, dest.reshape(-1), P)
    xg = xg3.reshape(P, HD)

    y = _run_gmm(xg, w13, w2, block_expert, block_eord, block_first,
                 block_nexte, nblocks)
    out = jnp.sum(topk_w[..., None] * y[dest].astype(jnp.float32),
                  axis=1)

    return out, logits


# speedup vs baseline: 2.4984x; 1.1030x over previous
import functools

import jax
import jax.numpy as jnp
from jax import lax
from jax.experimental import pallas as pl
from jax.experimental.pallas import tpu as pltpu
from jax.experimental.pallas import tpu_sc as plsc

NE = 64
KTOP = 8
HD = 2048
ID = 768
BT = 128
RB = 256


def _router_body(x_ref, gw_ref, logits_ref, wout_ref, iout_ref, cnt_ref):
    x = x_ref[...]
    logits = lax.dot_general(x, gw_ref[...], (((1,), (1,)), ((), ())),
                             preferred_element_type=jnp.float32)
    logits_ref[...] = logits
    m = jnp.max(logits, axis=-1, keepdims=True)
    p = jnp.exp(logits - m)
    probs = p / jnp.sum(p, axis=-1, keepdims=True)

    iot = lax.broadcasted_iota(jnp.int32, probs.shape, 1)
    ws = []
    idxs = []
    pc = probs
    for _ in range(KTOP):
        mx = jnp.max(pc, axis=-1, keepdims=True)
        cand = jnp.where(pc == mx, iot, NE)
        idx = jnp.min(cand, axis=-1, keepdims=True)
        ws.append(mx)
        idxs.append(idx)
        pc = jnp.where(iot == idx, -1.0, pc)
    w = jnp.concatenate(ws, axis=-1)
    w = w / jnp.sum(w, axis=-1, keepdims=True)
    wout_ref[...] = w
    idx = jnp.concatenate(idxs, axis=-1)
    iout_ref[...] = idx
    cnt = jnp.zeros((1, NE), jnp.float32)
    for k in range(KTOP):
        ohk = (idx[:, k:k + 1] ==
               lax.broadcasted_iota(jnp.int32, (RB, NE), 1)).astype(jnp.float32)
        cnt = cnt + jnp.sum(ohk, axis=0, keepdims=True)
    cnt_ref[...] = cnt.astype(jnp.int32)[None]


def _run_router(x, gate_w):
    T = x.shape[0]
    grid = (T // RB,)
    return pl.pallas_call(
        _router_body,
        grid=grid,
        in_specs=[
            pl.BlockSpec((RB, HD), lambda i: (i, 0)),
            pl.BlockSpec((NE, HD), lambda i: (0, 0)),
        ],
        out_specs=[
            pl.BlockSpec((RB, NE), lambda i: (i, 0)),
            pl.BlockSpec((RB, KTOP), lambda i: (i, 0)),
            pl.BlockSpec((RB, KTOP), lambda i: (i, 0)),
            pl.BlockSpec((1, 1, NE), lambda i: (i, 0, 0)),
        ],
        out_shape=[
            jax.ShapeDtypeStruct((T, NE), jnp.float32),
            jax.ShapeDtypeStruct((T, KTOP), jnp.float32),
            jax.ShapeDtypeStruct((T, KTOP), jnp.int32),
            jax.ShapeDtypeStruct((T // RB, 1, NE), jnp.int32),
        ],
    )(x, gate_w)


def _dest_body(i_ref, pb_ref, po_ref, dest_ref):
    idx = i_ref[...]
    base = (pb_ref[0] + po_ref[0]).astype(jnp.float32)
    tril = (lax.broadcasted_iota(jnp.int32, (RB, RB), 0) >
            lax.broadcasted_iota(jnp.int32, (RB, RB), 1)).astype(jnp.float32)
    iot = lax.broadcasted_iota(jnp.int32, (RB, NE), 1)
    for k in range(KTOP):
        ohk = (idx[:, k:k + 1] == iot).astype(jnp.float32)
        prev_t = lax.dot_general(tril, ohk, (((1,), (0,)), ((), ())),
                                 preferred_element_type=jnp.float32)
        slot = jnp.sum((prev_t + base) * ohk, axis=1)
        dest_ref[:, k] = slot.astype(jnp.int32)
        base = base + jnp.sum(ohk, axis=0, keepdims=True)


def _run_dest(topk_i, prefix_before, padded_off):
    T = topk_i.shape[0]
    return pl.pallas_call(
        _dest_body,
        grid=(T // RB,),
        in_specs=[
            pl.BlockSpec((RB, KTOP), lambda i: (i, 0)),
            pl.BlockSpec((1, 1, NE), lambda i: (i, 0, 0)),
            pl.BlockSpec((1, 1, NE), lambda i: (0, 0, 0)),
        ],
        out_specs=pl.BlockSpec((RB, KTOP), lambda i: (i, 0)),
        out_shape=jax.ShapeDtypeStruct((T, KTOP), jnp.int32),
    )(topk_i, prefix_before, padded_off)


SC_NW = 32
SC_C = 16


def _sc_dispatch_body(x2_hbm, tok_hbm, dest_hbm, xg2_hbm,
                      tok_v, dest_v, gbuf, gsems, ssems):
    wid = lax.axis_index("s") * 2 + lax.axis_index("c")
    TK = tok_hbm.shape[0]
    per_w = TK // SC_NW
    nch = per_w // SC_C
    base = wid * per_w
    pltpu.sync_copy(tok_hbm.at[pl.ds(base, per_w)], tok_v)
    pltpu.sync_copy(dest_hbm.at[pl.ds(base, per_w)], dest_v)
    pending = {}
    for j in range(nch):
        tvec = tok_v[pl.ds(j * SC_C, SC_C)]
        dvec = dest_v[pl.ds(j * SC_C, SC_C)]
        sl = j % 2
        if j >= 2:
            pending[j - 2].wait()
        g = pltpu.make_async_copy(x2_hbm.at[tvec], gbuf.at[sl], gsems[sl])
        g.start()
        g.wait()
        s = pltpu.make_async_copy(gbuf.at[sl], xg2_hbm.at[dvec], ssems[sl])
        s.start()
        pending[j] = s
    pending[nch - 2].wait()
    pending[nch - 1].wait()


def _run_sc_dispatch(x2, tok, dest, P):
    TK = tok.shape[0]
    mesh = plsc.VectorSubcoreMesh(core_axis_name="c", subcore_axis_name="s")
    f = pl.kernel(
        _sc_dispatch_body,
        out_type=jax.ShapeDtypeStruct((P, HD), jnp.float32),
        mesh=mesh,
        scratch_types=[
            pltpu.VMEM((TK // SC_NW,), jnp.int32),
            pltpu.VMEM((TK // SC_NW,), jnp.int32),
            pltpu.VMEM((2, SC_C, HD), jnp.float32),
            (pltpu.SemaphoreType.DMA, pltpu.SemaphoreType.DMA),
            (pltpu.SemaphoreType.DMA, pltpu.SemaphoreType.DMA),
        ],
    )
    return f(x2, tok, dest)


NC13 = 8
NC2 = 4
C13 = 2 * ID // NC13
C2 = HD // NC2


def _start_weight_dmas(e, slot, w13_hbm, w2_hbm, w13c, w2c, sem13, sem2):
    for c in range(NC13):
        pltpu.make_async_copy(
            w13_hbm.at[e, pl.ds(c * C13, C13)],
            w13c.at[slot, pl.ds(c * C13, C13)],
            sem13.at[slot, c]).start()
    for c in range(NC2):
        pltpu.make_async_copy(
            w2_hbm.at[e, pl.ds(c * C2, C2)],
            w2c.at[slot, pl.ds(c * C2, C2)],
            sem2.at[slot, c]).start()


def _wait_weight_dmas(e, slot, w13_hbm, w2_hbm, w13c, w2c, sem13, sem2):
    for c in range(NC13):
        pltpu.make_async_copy(
            w13_hbm.at[e, pl.ds(c * C13, C13)],
            w13c.at[slot, pl.ds(c * C13, C13)],
            sem13.at[slot, c]).wait()
    for c in range(NC2):
        pltpu.make_async_copy(
            w2_hbm.at[e, pl.ds(c * C2, C2)],
            w2c.at[slot, pl.ds(c * C2, C2)],
            sem2.at[slot, c]).wait()


def _gmm_body(be_ref, eord_ref, first_ref, nexte_ref,
              xg_ref, w13_hbm, w2_hbm, y_ref,
              w13c, w2c, w13b, w2b, sem13, sem2):
    b = pl.program_id(0)
    be = be_ref[b]

    @pl.when(be >= 0)
    def _():
        slot = eord_ref[b] & 1

        @pl.when(b == 0)
        def _():
            _start_weight_dmas(be, slot, w13_hbm, w2_hbm, w13c, w2c,
                               sem13, sem2)

        @pl.when(first_ref[b] == 1)
        def _():
            nxt = nexte_ref[b]

            @pl.when(nxt >= 0)
            def _():
                _start_weight_dmas(nxt, 1 - slot, w13_hbm, w2_hbm,
                                   w13c, w2c, sem13, sem2)

            _wait_weight_dmas(be, slot, w13_hbm, w2_hbm, w13c, w2c,
                              sem13, sem2)
            w13b[...] = w13c[slot].astype(jnp.bfloat16)
            w2b[...] = w2c[slot].astype(jnp.bfloat16)

        xb = xg_ref[...].astype(jnp.bfloat16)
        h = lax.dot_general(xb, w13b[...], (((1,), (1,)), ((), ())),
                            preferred_element_type=jnp.float32)
        g = h[:, :ID]
        u = h[:, ID:]
        a = (g * (1.0 / (1.0 + jnp.exp(-g))) * u).astype(jnp.bfloat16)
        y_ref[...] = lax.dot_general(a, w2b[...], (((1,), (1,)), ((), ())),
                                     preferred_element_type=jnp.float32
                                     ).astype(jnp.bfloat16)


def _run_gmm(xg, w13, w2, block_expert, block_eord, block_first, block_nexte,
             nblocks):
    P = xg.shape[0]
    grid_spec = pltpu.PrefetchScalarGridSpec(
        num_scalar_prefetch=4,
        grid=(nblocks,),
        in_specs=[
            pl.BlockSpec((BT, HD), lambda b, *_: (b, 0)),
            pl.BlockSpec(memory_space=pl.ANY),
            pl.BlockSpec(memory_space=pl.ANY),
        ],
        out_specs=pl.BlockSpec((BT, HD), lambda b, *_: (b, 0)),
        scratch_shapes=[
            pltpu.VMEM((2, 2 * ID, HD), jnp.float32),
            pltpu.VMEM((2, HD, ID), jnp.float32),
            pltpu.VMEM((2 * ID, HD), jnp.bfloat16),
            pltpu.VMEM((HD, ID), jnp.bfloat16),
            pltpu.SemaphoreType.DMA((2, NC13)),
            pltpu.SemaphoreType.DMA((2, NC2)),
        ],
    )
    return pl.pallas_call(
        _gmm_body,
        grid_spec=grid_spec,
        out_shape=jax.ShapeDtypeStruct((P, HD), jnp.bfloat16),
    )(block_expert, block_eord, block_first, block_nexte, xg, w13, w2)


def _build_routing(counts_blk, TK):
    nblocks = TK // BT + NE
    counts = jnp.sum(counts_blk, axis=0)
    prefix_before = jnp.cumsum(counts_blk, axis=0) - counts_blk
    nb_e = (counts + BT - 1) // BT
    cum_nb = jnp.cumsum(nb_e)
    total_blocks = cum_nb[-1]
    padded_off = BT * (cum_nb - nb_e)
    barr = jnp.arange(nblocks, dtype=jnp.int32)
    block_expert = jnp.where(
        barr < total_blocks,
        jnp.searchsorted(cum_nb, barr, side='right').astype(jnp.int32),
        -1)

    used = counts > 0
    ord_e = jnp.cumsum(used.astype(jnp.int32)) - 1
    ord_clamped = jnp.where(used, ord_e, NE)
    expert_of_ord = jnp.full((NE + 1,), -1, jnp.int32).at[ord_clamped].set(
        jnp.arange(NE, dtype=jnp.int32), mode='drop')
    next_of_e = expert_of_ord[jnp.clip(ord_e + 1, 0, NE)]
    live = block_expert >= 0
    e_safe = jnp.maximum(block_expert, 0)
    block_eord = jnp.where(live, ord_e[e_safe], 0).astype(jnp.int32)
    block_nexte = jnp.where(live, next_of_e[e_safe], -1).astype(jnp.int32)
    prev_e = jnp.concatenate([jnp.array([-2], jnp.int32), block_expert[:-1]])
    block_first = (live & (block_expert != prev_e)).astype(jnp.int32)
    return (prefix_before, padded_off, block_expert,
            block_eord, block_first, block_nexte)


@jax.jit
def kernel(hidden_states, gate_w, w13, w2):
    B, S, Hd = hidden_states.shape
    x = hidden_states.reshape(-1, Hd)
    T = x.shape[0]
    TK = T * KTOP

    logits, topk_w, topk_i, counts_blk3 = _run_router(x, gate_w)
    (prefix_before, padded_off, block_expert,
     block_eord, block_first, block_nexte) = _build_routing(
         counts_blk3[:, 0, :], TK)

    dest = _run_dest(topk_i, prefix_before[:, None, :],
                     padded_off[None, None, :])
    nblocks = TK // BT + NE
    P = nblocks * BT

    tok = (jnp.arange(TK, dtype=jnp.int32) // KTOP)
    xg = _run_sc_dispatch(x, tok, dest.reshape(-1), P)

    y = _run_gmm(xg, w13, w2, block_expert, block_eord, block_first,
                 block_nexte, nblocks)
    out = jnp.sum(topk_w[..., None] * y[dest].astype(jnp.float32),
                  axis=1)

    return out, logits


# speedup vs baseline: 2.5042x; 1.0023x over previous
import functools

import jax
import jax.numpy as jnp
from jax import lax
from jax.experimental import pallas as pl
from jax.experimental.pallas import tpu as pltpu
from jax.experimental.pallas import tpu_sc as plsc

NE = 64
KTOP = 8
HD = 2048
ID = 768
BT = 128
RB = 256


def _router_body(x_ref, gw_ref, logits_ref, wout_ref, iout_ref, cnt_ref):
    x = x_ref[...]
    logits = lax.dot_general(x, gw_ref[...], (((1,), (1,)), ((), ())),
                             preferred_element_type=jnp.float32)
    logits_ref[...] = logits
    m = jnp.max(logits, axis=-1, keepdims=True)
    p = jnp.exp(logits - m)
    probs = p / jnp.sum(p, axis=-1, keepdims=True)

    iot = lax.broadcasted_iota(jnp.int32, probs.shape, 1)
    ws = []
    idxs = []
    pc = probs
    for _ in range(KTOP):
        mx = jnp.max(pc, axis=-1, keepdims=True)
        cand = jnp.where(pc == mx, iot, NE)
        idx = jnp.min(cand, axis=-1, keepdims=True)
        ws.append(mx)
        idxs.append(idx)
        pc = jnp.where(iot == idx, -1.0, pc)
    w = jnp.concatenate(ws, axis=-1)
    w = w / jnp.sum(w, axis=-1, keepdims=True)
    wout_ref[...] = w
    idx = jnp.concatenate(idxs, axis=-1)
    iout_ref[...] = idx
    cnt = jnp.zeros((1, NE), jnp.float32)
    for k in range(KTOP):
        ohk = (idx[:, k:k + 1] ==
               lax.broadcasted_iota(jnp.int32, (RB, NE), 1)).astype(jnp.float32)
        cnt = cnt + jnp.sum(ohk, axis=0, keepdims=True)
    cnt_ref[...] = cnt.astype(jnp.int32)[None]


def _run_router(x, gate_w):
    T = x.shape[0]
    grid = (T // RB,)
    return pl.pallas_call(
        _router_body,
        grid=grid,
        in_specs=[
            pl.BlockSpec((RB, HD), lambda i: (i, 0)),
            pl.BlockSpec((NE, HD), lambda i: (0, 0)),
        ],
        out_specs=[
            pl.BlockSpec((RB, NE), lambda i: (i, 0)),
            pl.BlockSpec((RB, KTOP), lambda i: (i, 0)),
            pl.BlockSpec((RB, KTOP), lambda i: (i, 0)),
            pl.BlockSpec((1, 1, NE), lambda i: (i, 0, 0)),
        ],
        out_shape=[
            jax.ShapeDtypeStruct((T, NE), jnp.float32),
            jax.ShapeDtypeStruct((T, KTOP), jnp.float32),
            jax.ShapeDtypeStruct((T, KTOP), jnp.int32),
            jax.ShapeDtypeStruct((T // RB, 1, NE), jnp.int32),
        ],
    )(x, gate_w)


def _dest_body(i_ref, pb_ref, po_ref, dest_ref):
    idx = i_ref[...]
    base = (pb_ref[0] + po_ref[0]).astype(jnp.float32)
    tril = (lax.broadcasted_iota(jnp.int32, (RB, RB), 0) >
            lax.broadcasted_iota(jnp.int32, (RB, RB), 1)).astype(jnp.float32)
    iot = lax.broadcasted_iota(jnp.int32, (RB, NE), 1)
    for k in range(KTOP):
        ohk = (idx[:, k:k + 1] == iot).astype(jnp.float32)
        prev_t = lax.dot_general(tril, ohk, (((1,), (0,)), ((), ())),
                                 preferred_element_type=jnp.float32)
        slot = jnp.sum((prev_t + base) * ohk, axis=1)
        dest_ref[:, k] = slot.astype(jnp.int32)
        base = base + jnp.sum(ohk, axis=0, keepdims=True)


def _run_dest(topk_i, prefix_before, padded_off):
    T = topk_i.shape[0]
    return pl.pallas_call(
        _dest_body,
        grid=(T // RB,),
        in_specs=[
            pl.BlockSpec((RB, KTOP), lambda i: (i, 0)),
            pl.BlockSpec((1, 1, NE), lambda i: (i, 0, 0)),
            pl.BlockSpec((1, 1, NE), lambda i: (0, 0, 0)),
        ],
        out_specs=pl.BlockSpec((RB, KTOP), lambda i: (i, 0)),
        out_shape=jax.ShapeDtypeStruct((T, KTOP), jnp.int32),
    )(topk_i, prefix_before, padded_off)


SC_NW = 32
SC_C = 16


def _sc_dispatch_body(x2_hbm, tok_hbm, dest_hbm, xg2_hbm,
                      tok_v, dest_v, gbuf, gsems, ssems):
    wid = lax.axis_index("s") * 2 + lax.axis_index("c")
    TK = tok_hbm.shape[0]
    per_w = TK // SC_NW
    nch = per_w // SC_C
    base = wid * per_w
    pltpu.sync_copy(tok_hbm.at[pl.ds(base, per_w)], tok_v)
    pltpu.sync_copy(dest_hbm.at[pl.ds(base, per_w)], dest_v)
    NB_ = 3
    gathers, scatters = {}, {}

    def start_gather(j):
        tvec = tok_v[pl.ds(j * SC_C, SC_C)]
        g = pltpu.make_async_copy(x2_hbm.at[tvec], gbuf.at[j % NB_],
                                  gsems[j % NB_])
        g.start()
        gathers[j] = g

    for j in range(min(2, nch)):
        start_gather(j)
    for j in range(nch):
        if j + 2 < nch:
            if j + 2 >= NB_:
                scatters[j + 2 - NB_].wait()
            start_gather(j + 2)
        gathers[j].wait()
        dvec = dest_v[pl.ds(j * SC_C, SC_C)]
        s = pltpu.make_async_copy(gbuf.at[j % NB_], xg2_hbm.at[dvec],
                                  ssems[j % NB_])
        s.start()
        scatters[j] = s
    for j in range(max(0, nch - NB_), nch):
        scatters[j].wait()


def _run_sc_dispatch(x2, tok, dest, P):
    TK = tok.shape[0]
    mesh = plsc.VectorSubcoreMesh(core_axis_name="c", subcore_axis_name="s")
    f = pl.kernel(
        _sc_dispatch_body,
        out_type=jax.ShapeDtypeStruct((P, HD), jnp.float32),
        mesh=mesh,
        scratch_types=[
            pltpu.VMEM((TK // SC_NW,), jnp.int32),
            pltpu.VMEM((TK // SC_NW,), jnp.int32),
            pltpu.VMEM((3, SC_C, HD), jnp.float32),
            (pltpu.SemaphoreType.DMA,) * 3,
            (pltpu.SemaphoreType.DMA,) * 3,
        ],
    )
    return f(x2, tok, dest)


NC13 = 8
NC2 = 4
C13 = 2 * ID // NC13
C2 = HD // NC2


def _start_weight_dmas(e, slot, w13_hbm, w2_hbm, w13c, w2c, sem13, sem2):
    for c in range(NC13):
        pltpu.make_async_copy(
            w13_hbm.at[e, pl.ds(c * C13, C13)],
            w13c.at[slot, pl.ds(c * C13, C13)],
            sem13.at[slot, c]).start()
    for c in range(NC2):
        pltpu.make_async_copy(
            w2_hbm.at[e, pl.ds(c * C2, C2)],
            w2c.at[slot, pl.ds(c * C2, C2)],
            sem2.at[slot, c]).start()


def _wait_weight_dmas(e, slot, w13_hbm, w2_hbm, w13c, w2c, sem13, sem2):
    for c in range(NC13):
        pltpu.make_async_copy(
            w13_hbm.at[e, pl.ds(c * C13, C13)],
            w13c.at[slot, pl.ds(c * C13, C13)],
            sem13.at[slot, c]).wait()
    for c in range(NC2):
        pltpu.make_async_copy(
            w2_hbm.at[e, pl.ds(c * C2, C2)],
            w2c.at[slot, pl.ds(c * C2, C2)],
            sem2.at[slot, c]).wait()


def _gmm_body(be_ref, eord_ref, first_ref, nexte_ref,
              xg_ref, w13_hbm, w2_hbm, y_ref,
              w13c, w2c, w13b, w2b, sem13, sem2):
    b = pl.program_id(0)
    be = be_ref[b]

    @pl.when(be >= 0)
    def _():
        slot = eord_ref[b] & 1

        @pl.when(b == 0)
        def _():
            _start_weight_dmas(be, slot, w13_hbm, w2_hbm, w13c, w2c,
                               sem13, sem2)

        @pl.when(first_ref[b] == 1)
        def _():
            nxt = nexte_ref[b]

            @pl.when(nxt >= 0)
            def _():
                _start_weight_dmas(nxt, 1 - slot, w13_hbm, w2_hbm,
                                   w13c, w2c, sem13, sem2)

            _wait_weight_dmas(be, slot, w13_hbm, w2_hbm, w13c, w2c,
                              sem13, sem2)
            w13b[...] = w13c[slot].astype(jnp.bfloat16)
            w2b[...] = w2c[slot].astype(jnp.bfloat16)

        xb = xg_ref[...].astype(jnp.bfloat16)
        h = lax.dot_general(xb, w13b[...], (((1,), (1,)), ((), ())),
                            preferred_element_type=jnp.float32)
        g = h[:, :ID]
        u = h[:, ID:]
        a = (g * (1.0 / (1.0 + jnp.exp(-g))) * u).astype(jnp.bfloat16)
        y_ref[...] = lax.dot_general(a, w2b[...], (((1,), (1,)), ((), ())),
                                     preferred_element_type=jnp.float32
                                     ).astype(jnp.bfloat16)


def _run_gmm(xg, w13, w2, block_expert, block_eord, block_first, block_nexte,
             nblocks):
    P = xg.shape[0]
    grid_spec = pltpu.PrefetchScalarGridSpec(
        num_scalar_prefetch=4,
        grid=(nblocks,),
        in_specs=[
            pl.BlockSpec((BT, HD), lambda b, *_: (b, 0)),
            pl.BlockSpec(memory_space=pl.ANY),
            pl.BlockSpec(memory_space=pl.ANY),
        ],
        out_specs=pl.BlockSpec((BT, HD), lambda b, *_: (b, 0)),
        scratch_shapes=[
            pltpu.VMEM((2, 2 * ID, HD), jnp.float32),
            pltpu.VMEM((2, HD, ID), jnp.float32),
            pltpu.VMEM((2 * ID, HD), jnp.bfloat16),
            pltpu.VMEM((HD, ID), jnp.bfloat16),
            pltpu.SemaphoreType.DMA((2, NC13)),
            pltpu.SemaphoreType.DMA((2, NC2)),
        ],
    )
    return pl.pallas_call(
        _gmm_body,
        grid_spec=grid_spec,
        out_shape=jax.ShapeDtypeStruct((P, HD), jnp.bfloat16),
    )(block_expert, block_eord, block_first, block_nexte, xg, w13, w2)


def _build_routing(counts_blk, TK):
    nblocks = TK // BT + NE
    counts = jnp.sum(counts_blk, axis=0)
    prefix_before = jnp.cumsum(counts_blk, axis=0) - counts_blk
    nb_e = (counts + BT - 1) // BT
    cum_nb = jnp.cumsum(nb_e)
    total_blocks = cum_nb[-1]
    padded_off = BT * (cum_nb - nb_e)
    barr = jnp.arange(nblocks, dtype=jnp.int32)
    block_expert = jnp.where(
        barr < total_blocks,
        jnp.searchsorted(cum_nb, barr, side='right').astype(jnp.int32),
        -1)

    used = counts > 0
    ord_e = jnp.cumsum(used.astype(jnp.int32)) - 1
    ord_clamped = jnp.where(used, ord_e, NE)
    expert_of_ord = jnp.full((NE + 1,), -1, jnp.int32).at[ord_clamped].set(
        jnp.arange(NE, dtype=jnp.int32), mode='drop')
    next_of_e = expert_of_ord[jnp.clip(ord_e + 1, 0, NE)]
    live = block_expert >= 0
    e_safe = jnp.maximum(block_expert, 0)
    block_eord = jnp.where(live, ord_e[e_safe], 0).astype(jnp.int32)
    block_nexte = jnp.where(live, next_of_e[e_safe], -1).astype(jnp.int32)
    prev_e = jnp.concatenate([jnp.array([-2], jnp.int32), block_expert[:-1]])
    block_first = (live & (block_expert != prev_e)).astype(jnp.int32)
    return (prefix_before, padded_off, block_expert,
            block_eord, block_first, block_nexte)


@jax.jit
def kernel(hidden_states, gate_w, w13, w2):
    B, S, Hd = hidden_states.shape
    x = hidden_states.reshape(-1, Hd)
    T = x.shape[0]
    TK = T * KTOP

    logits, topk_w, topk_i, counts_blk3 = _run_router(x, gate_w)
    (prefix_before, padded_off, block_expert,
     block_eord, block_first, block_nexte) = _build_routing(
         counts_blk3[:, 0, :], TK)

    dest = _run_dest(topk_i, prefix_before[:, None, :],
                     padded_off[None, None, :])
    nblocks = TK // BT + NE
    P = nblocks * BT

    tok = (jnp.arange(TK, dtype=jnp.int32) // KTOP)
    xg = _run_sc_dispatch(x, tok, dest.reshape(-1), P)

    y = _run_gmm(xg, w13, w2, block_expert, block_eord, block_first,
                 block_nexte, nblocks)
    out = jnp.sum(topk_w[..., None] * y[dest].astype(jnp.float32),
                  axis=1)

    return out, logits


# speedup vs baseline: 2.5528x; 1.0194x over previous
import functools

import jax
import jax.numpy as jnp
from jax import lax
from jax.experimental import pallas as pl
from jax.experimental.pallas import tpu as pltpu
from jax.experimental.pallas import tpu_sc as plsc

NE = 64
KTOP = 8
HD = 2048
ID = 768
BT = 128
RB = 256


def _router_body(x_ref, gw_ref, logits_ref, wout_ref, iout_ref, cnt_ref,
                 xpk_ref):
    x = x_ref[...]
    ia = lax.bitcast_convert_type(x[:, :HD // 2], jnp.uint32)
    ib = lax.bitcast_convert_type(x[:, HD // 2:], jnp.uint32)
    ra = (ia + 0x7FFF + ((ia >> 16) & 1)) >> 16
    rb = (ib + 0x7FFF + ((ib >> 16) & 1)) >> 16
    xpk_ref[...] = ((ra & 0xFFFF) | (rb << 16)).astype(jnp.int32)
    logits = lax.dot_general(x, gw_ref[...], (((1,), (1,)), ((), ())),
                             preferred_element_type=jnp.float32)
    logits_ref[...] = logits
    m = jnp.max(logits, axis=-1, keepdims=True)
    p = jnp.exp(logits - m)
    probs = p / jnp.sum(p, axis=-1, keepdims=True)

    iot = lax.broadcasted_iota(jnp.int32, probs.shape, 1)
    ws = []
    idxs = []
    pc = probs
    for _ in range(KTOP):
        mx = jnp.max(pc, axis=-1, keepdims=True)
        cand = jnp.where(pc == mx, iot, NE)
        idx = jnp.min(cand, axis=-1, keepdims=True)
        ws.append(mx)
        idxs.append(idx)
        pc = jnp.where(iot == idx, -1.0, pc)
    w = jnp.concatenate(ws, axis=-1)
    w = w / jnp.sum(w, axis=-1, keepdims=True)
    wout_ref[...] = w
    idx = jnp.concatenate(idxs, axis=-1)
    iout_ref[...] = idx
    cnt = jnp.zeros((1, NE), jnp.float32)
    for k in range(KTOP):
        ohk = (idx[:, k:k + 1] ==
               lax.broadcasted_iota(jnp.int32, (RB, NE), 1)).astype(jnp.float32)
        cnt = cnt + jnp.sum(ohk, axis=0, keepdims=True)
    cnt_ref[...] = cnt.astype(jnp.int32)[None]


def _run_router(x, gate_w):
    T = x.shape[0]
    grid = (T // RB,)
    return pl.pallas_call(
        _router_body,
        grid=grid,
        in_specs=[
            pl.BlockSpec((RB, HD), lambda i: (i, 0)),
            pl.BlockSpec((NE, HD), lambda i: (0, 0)),
        ],
        out_specs=[
            pl.BlockSpec((RB, NE), lambda i: (i, 0)),
            pl.BlockSpec((RB, KTOP), lambda i: (i, 0)),
            pl.BlockSpec((RB, KTOP), lambda i: (i, 0)),
            pl.BlockSpec((1, 1, NE), lambda i: (i, 0, 0)),
            pl.BlockSpec((RB, HD // 2), lambda i: (i, 0)),
        ],
        out_shape=[
            jax.ShapeDtypeStruct((T, NE), jnp.float32),
            jax.ShapeDtypeStruct((T, KTOP), jnp.float32),
            jax.ShapeDtypeStruct((T, KTOP), jnp.int32),
            jax.ShapeDtypeStruct((T // RB, 1, NE), jnp.int32),
            jax.ShapeDtypeStruct((T, HD // 2), jnp.int32),
        ],
    )(x, gate_w)


def _dest_body(i_ref, pb_ref, po_ref, dest_ref):
    idx = i_ref[...]
    base = (pb_ref[0] + po_ref[0]).astype(jnp.float32)
    tril = (lax.broadcasted_iota(jnp.int32, (RB, RB), 0) >
            lax.broadcasted_iota(jnp.int32, (RB, RB), 1)).astype(jnp.float32)
    iot = lax.broadcasted_iota(jnp.int32, (RB, NE), 1)
    for k in range(KTOP):
        ohk = (idx[:, k:k + 1] == iot).astype(jnp.float32)
        prev_t = lax.dot_general(tril, ohk, (((1,), (0,)), ((), ())),
                                 preferred_element_type=jnp.float32)
        slot = jnp.sum((prev_t + base) * ohk, axis=1)
        dest_ref[:, k] = slot.astype(jnp.int32)
        base = base + jnp.sum(ohk, axis=0, keepdims=True)


def _run_dest(topk_i, prefix_before, padded_off):
    T = topk_i.shape[0]
    return pl.pallas_call(
        _dest_body,
        grid=(T // RB,),
        in_specs=[
            pl.BlockSpec((RB, KTOP), lambda i: (i, 0)),
            pl.BlockSpec((1, 1, NE), lambda i: (i, 0, 0)),
            pl.BlockSpec((1, 1, NE), lambda i: (0, 0, 0)),
        ],
        out_specs=pl.BlockSpec((RB, KTOP), lambda i: (i, 0)),
        out_shape=jax.ShapeDtypeStruct((T, KTOP), jnp.int32),
    )(topk_i, prefix_before, padded_off)


SC_NW = 32
SC_C = 16


def _sc_dispatch_body(x2_hbm, tok_hbm, dest_hbm, xg2_hbm,
                      tok_v, dest_v, gbuf, gsems, ssems):
    wid = lax.axis_index("s") * 2 + lax.axis_index("c")
    TK = tok_hbm.shape[0]
    per_w = TK // SC_NW
    nch = per_w // SC_C
    base = wid * per_w
    pltpu.sync_copy(tok_hbm.at[pl.ds(base, per_w)], tok_v)
    pltpu.sync_copy(dest_hbm.at[pl.ds(base, per_w)], dest_v)
    NB_ = 3
    gathers, scatters = {}, {}

    def start_gather(j):
        tvec = tok_v[pl.ds(j * SC_C, SC_C)]
        g = pltpu.make_async_copy(x2_hbm.at[tvec], gbuf.at[j % NB_],
                                  gsems[j % NB_])
        g.start()
        gathers[j] = g

    for j in range(min(2, nch)):
        start_gather(j)
    for j in range(nch):
        if j + 2 < nch:
            if j + 2 >= NB_:
                scatters[j + 2 - NB_].wait()
            start_gather(j + 2)
        gathers[j].wait()
        dvec = dest_v[pl.ds(j * SC_C, SC_C)]
        s = pltpu.make_async_copy(gbuf.at[j % NB_], xg2_hbm.at[dvec],
                                  ssems[j % NB_])
        s.start()
        scatters[j] = s
    for j in range(max(0, nch - NB_), nch):
        scatters[j].wait()


def _run_sc_dispatch(x2, tok, dest, P):
    TK = tok.shape[0]
    mesh = plsc.VectorSubcoreMesh(core_axis_name="c", subcore_axis_name="s")
    f = pl.kernel(
        _sc_dispatch_body,
        out_type=jax.ShapeDtypeStruct((P, HD // 2), jnp.int32),
        mesh=mesh,
        scratch_types=[
            pltpu.VMEM((TK // SC_NW,), jnp.int32),
            pltpu.VMEM((TK // SC_NW,), jnp.int32),
            pltpu.VMEM((3, SC_C, HD // 2), jnp.int32),
            (pltpu.SemaphoreType.DMA,) * 3,
            (pltpu.SemaphoreType.DMA,) * 3,
        ],
    )
    return f(x2, tok, dest)


NC13 = 8
NC2 = 4
C13 = 2 * ID // NC13
C2 = HD // NC2


def _start_weight_dmas(e, slot, w13_hbm, w2_hbm, w13c, w2c, sem13, sem2):
    for c in range(NC13):
        pltpu.make_async_copy(
            w13_hbm.at[e, pl.ds(c * C13, C13)],
            w13c.at[slot, pl.ds(c * C13, C13)],
            sem13.at[slot, c]).start()
    for c in range(NC2):
        pltpu.make_async_copy(
            w2_hbm.at[e, pl.ds(c * C2, C2)],
            w2c.at[slot, pl.ds(c * C2, C2)],
            sem2.at[slot, c]).start()


def _wait_weight_dmas(e, slot, w13_hbm, w2_hbm, w13c, w2c, sem13, sem2):
    for c in range(NC13):
        pltpu.make_async_copy(
            w13_hbm.at[e, pl.ds(c * C13, C13)],
            w13c.at[slot, pl.ds(c * C13, C13)],
            sem13.at[slot, c]).wait()
    for c in range(NC2):
        pltpu.make_async_copy(
            w2_hbm.at[e, pl.ds(c * C2, C2)],
            w2c.at[slot, pl.ds(c * C2, C2)],
            sem2.at[slot, c]).wait()


def _gmm_body(be_ref, eord_ref, first_ref, nexte_ref,
              xg_ref, w13_hbm, w2_hbm, y_ref,
              w13c, w2c, w13b, w2b, sem13, sem2):
    b = pl.program_id(0)
    be = be_ref[b]

    @pl.when(be >= 0)
    def _():
        slot = eord_ref[b] & 1

        @pl.when(b == 0)
        def _():
            _start_weight_dmas(be, slot, w13_hbm, w2_hbm, w13c, w2c,
                               sem13, sem2)

        @pl.when(first_ref[b] == 1)
        def _():
            nxt = nexte_ref[b]

            @pl.when(nxt >= 0)
            def _():
                _start_weight_dmas(nxt, 1 - slot, w13_hbm, w2_hbm,
                                   w13c, w2c, sem13, sem2)

            _wait_weight_dmas(be, slot, w13_hbm, w2_hbm, w13c, w2c,
                              sem13, sem2)
            w13b[...] = w13c[slot].astype(jnp.bfloat16)
            w2b[...] = w2c[slot].astype(jnp.bfloat16)

        v = lax.bitcast_convert_type(xg_ref[...], jnp.uint32)
        flo = lax.bitcast_convert_type(v << 16, jnp.float32)
        fhi = lax.bitcast_convert_type(v & jnp.uint32(0xFFFF0000),
                                       jnp.float32)
        xb = jnp.concatenate([flo, fhi], axis=1).astype(jnp.bfloat16)
        h = lax.dot_general(xb, w13b[...], (((1,), (1,)), ((), ())),
                            preferred_element_type=jnp.float32)
        g = h[:, :ID]
        u = h[:, ID:]
        a = (g * (1.0 / (1.0 + jnp.exp(-g))) * u).astype(jnp.bfloat16)
        y_ref[...] = lax.dot_general(a, w2b[...], (((1,), (1,)), ((), ())),
                                     preferred_element_type=jnp.float32
                                     ).astype(jnp.bfloat16)


def _run_gmm(xg, w13, w2, block_expert, block_eord, block_first, block_nexte,
             nblocks):
    P = xg.shape[0]
    grid_spec = pltpu.PrefetchScalarGridSpec(
        num_scalar_prefetch=4,
        grid=(nblocks,),
        in_specs=[
            pl.BlockSpec((BT, HD // 2), lambda b, *_: (b, 0)),
            pl.BlockSpec(memory_space=pl.ANY),
            pl.BlockSpec(memory_space=pl.ANY),
        ],
        out_specs=pl.BlockSpec((BT, HD), lambda b, *_: (b, 0)),
        scratch_shapes=[
            pltpu.VMEM((2, 2 * ID, HD), jnp.float32),
            pltpu.VMEM((2, HD, ID), jnp.float32),
            pltpu.VMEM((2 * ID, HD), jnp.bfloat16),
            pltpu.VMEM((HD, ID), jnp.bfloat16),
            pltpu.SemaphoreType.DMA((2, NC13)),
            pltpu.SemaphoreType.DMA((2, NC2)),
        ],
    )
    return pl.pallas_call(
        _gmm_body,
        grid_spec=grid_spec,
        out_shape=jax.ShapeDtypeStruct((P, HD), jnp.bfloat16),
    )(block_expert, block_eord, block_first, block_nexte, xg, w13, w2)


def _build_routing(counts_blk, TK):
    nblocks = TK // BT + NE
    counts = jnp.sum(counts_blk, axis=0)
    prefix_before = jnp.cumsum(counts_blk, axis=0) - counts_blk
    nb_e = (counts + BT - 1) // BT
    cum_nb = jnp.cumsum(nb_e)
    total_blocks = cum_nb[-1]
    padded_off = BT * (cum_nb - nb_e)
    barr = jnp.arange(nblocks, dtype=jnp.int32)
    block_expert = jnp.where(
        barr < total_blocks,
        jnp.searchsorted(cum_nb, barr, side='right').astype(jnp.int32),
        -1)

    used = counts > 0
    ord_e = jnp.cumsum(used.astype(jnp.int32)) - 1
    ord_clamped = jnp.where(used, ord_e, NE)
    expert_of_ord = jnp.full((NE + 1,), -1, jnp.int32).at[ord_clamped].set(
        jnp.arange(NE, dtype=jnp.int32), mode='drop')
    next_of_e = expert_of_ord[jnp.clip(ord_e + 1, 0, NE)]
    live = block_expert >= 0
    e_safe = jnp.maximum(block_expert, 0)
    block_eord = jnp.where(live, ord_e[e_safe], 0).astype(jnp.int32)
    block_nexte = jnp.where(live, next_of_e[e_safe], -1).astype(jnp.int32)
    prev_e = jnp.concatenate([jnp.array([-2], jnp.int32), block_expert[:-1]])
    block_first = (live & (block_expert != prev_e)).astype(jnp.int32)
    return (prefix_before, padded_off, block_expert,
            block_eord, block_first, block_nexte)


@jax.jit
def kernel(hidden_states, gate_w, w13, w2):
    B, S, Hd = hidden_states.shape
    x = hidden_states.reshape(-1, Hd)
    T = x.shape[0]
    TK = T * KTOP

    logits, topk_w, topk_i, counts_blk3, xpk = _run_router(x, gate_w)
    (prefix_before, padded_off, block_expert,
     block_eord, block_first, block_nexte) = _build_routing(
         counts_blk3[:, 0, :], TK)

    dest = _run_dest(topk_i, prefix_before[:, None, :],
                     padded_off[None, None, :])
    nblocks = TK // BT + NE
    P = nblocks * BT

    tok = (jnp.arange(TK, dtype=jnp.int32) // KTOP)
    xg = _run_sc_dispatch(xpk, tok, dest.reshape(-1), P)

    y = _run_gmm(xg, w13, w2, block_expert, block_eord, block_first,
                 block_nexte, nblocks)
    out = jnp.sum(topk_w[..., None] * y[dest].astype(jnp.float32),
                  axis=1)

    return out, logits
